# baseline (speedup 1.0000x reference)
import functools

import jax
import jax.numpy as jnp
from jax.experimental import pallas as pl
from jax.experimental.pallas import tpu as pltpu

_D = 1024
_F = 8192
_T = 2048
_K = 50

_TB = 256
_FB = 1024
_NT = _T // _TB
_NF = _F // _FB


def _norms_body(wdec_ref, sn_ref, rn_ref, wb_ref):
    w = wdec_ref[...]
    sq = jnp.sum(w * w, axis=1)[None, :]
    sn_ref[...] = jnp.sqrt(sq)
    rn_ref[...] = jnp.where(sq > 0.0, jax.lax.rsqrt(sq), 0.0)
    wb_ref[...] = w.astype(jnp.bfloat16)


def _fused_body(k_ref, hid_ref, wenc_ref, sn_ref, rn_ref, wdec_ref, b_ref,
                out_ref, g_scr, sp_scr):
    t = pl.program_id(0)
    f = pl.program_id(1)

    @pl.when(t > 0)
    def _decode():
        acc = jnp.dot(sp_scr[:, pl.ds(f * _FB, _FB)],
                      wdec_ref[pl.ds(f * _FB, _FB), :],
                      preferred_element_type=jnp.float32)

        @pl.when(f == 0)
        def _init():
            out_ref[...] = acc

        @pl.when(f != 0)
        def _accum():
            out_ref[...] += acc

        @pl.when(f == _NF - 1)
        def _bias():
            out_ref[...] += b_ref[...]

    @pl.when(t < _NT)
    def _encode():
        a = jnp.maximum(jnp.dot(hid_ref[...], wenc_ref[...],
                                preferred_element_type=jnp.float32), 0.0)
        g_scr[:, pl.ds(f * _FB, _FB)] = a * sn_ref[...]

    @pl.when((t < _NT) & (f == _NF - 1))
    def _select():
        kk = jnp.minimum(k_ref[0], _K)
        g = g_scr[...]
        lo0 = jnp.zeros((_TB, 1), jnp.int32)
        hi0 = jax.lax.bitcast_convert_type(
            jnp.max(g, axis=1, keepdims=True), jnp.int32)
        clo0 = jnp.full((_TB, 1), _F, jnp.int32)
        chi0 = jnp.ones((_TB, 1), jnp.int32)
        lk = jnp.log(jnp.maximum(kk, 1).astype(jnp.float32))

        def done(lo, hi, clo):
            return (clo == kk) | (hi - lo <= 1)

        def cond(carry):
            lo, hi, clo, chi = carry
            return jnp.logical_not(jnp.all(done(lo, hi, clo)))

        def body(carry):
            lo, hi, clo, chi = carry
            bise = lo + jax.lax.div(hi - lo, 2)
            vlo = jax.lax.bitcast_convert_type(lo, jnp.float32)
            vhi = jax.lax.bitcast_convert_type(hi, jnp.float32)
            llo = jnp.log(clo.astype(jnp.float32))
            lhi = jnp.log(jnp.maximum(chi, 1).astype(jnp.float32))
            frac = (llo - lk) / jnp.maximum(llo - lhi, 1e-6)
            vmid = vlo + (vhi - vlo) * frac
            imid = jnp.clip(jax.lax.bitcast_convert_type(vmid, jnp.int32),
                            lo + 1, hi - 1)
            m1 = jnp.minimum(imid, bise)
            m2 = jnp.maximum(imid, bise)
            v1 = jax.lax.bitcast_convert_type(m1, jnp.float32)
            v2 = jax.lax.bitcast_convert_type(m2, jnp.float32)
            c1 = jnp.sum((g >= v1).astype(jnp.int32), axis=1, keepdims=True)
            c2 = jnp.sum((g >= v2).astype(jnp.int32), axis=1, keepdims=True)
            ok2 = c2 >= kk
            ok1 = c1 >= kk
            nlo = jnp.where(ok2, m2, jnp.where(ok1, m1, lo))
            nclo = jnp.where(ok2, c2, jnp.where(ok1, c1, clo))
            nhi = jnp.where(ok2, hi, jnp.where(ok1, m2, m1))
            nchi = jnp.where(ok2, chi, jnp.where(ok1, c2, c1))
            return (nlo, nhi, nclo, nchi)

        carry = jax.lax.fori_loop(0, 13, lambda _, c: body(c),
                                  (lo0, hi0, clo0, chi0))
        lo, _, _, _ = jax.lax.while_loop(cond, body, carry)
        vthr = jax.lax.bitcast_convert_type(lo, jnp.float32)
        sp_scr[...] = jnp.where(g >= vthr, g * rn_ref[...],
                                0.0).astype(jnp.bfloat16)


@functools.partial(jax.jit, static_argnames=())
def kernel(hidden, W_enc, W_dec, b_dec, k):
    k_arr = jnp.asarray(k, jnp.int32).reshape((1,))

    sqrtn, rsqrtn, wdec_bf16 = pl.pallas_call(
        _norms_body,
        grid=(_NF,),
        in_specs=[pl.BlockSpec((_FB, _D), lambda f: (f, 0))],
        out_specs=[pl.BlockSpec((1, _FB), lambda f: (0, f)),
                   pl.BlockSpec((1, _FB), lambda f: (0, f)),
                   pl.BlockSpec((_FB, _D), lambda f: (f, 0))],
        out_shape=[jax.ShapeDtypeStruct((1, _F), jnp.float32),
                   jax.ShapeDtypeStruct((1, _F), jnp.float32),
                   jax.ShapeDtypeStruct((_F, _D), jnp.bfloat16)],
    )(W_dec)

    recon = pl.pallas_call(
        _fused_body,
        grid=(_NT + 1, _NF),
        in_specs=[
            pl.BlockSpec(memory_space=pltpu.SMEM),
            pl.BlockSpec((_TB, _D), lambda t, f: (jnp.minimum(t, _NT - 1), 0)),
            pl.BlockSpec((_D, _FB), lambda t, f: (0, f)),
            pl.BlockSpec((1, _FB), lambda t, f: (0, f)),
            pl.BlockSpec((1, _F), lambda t, f: (0, 0)),
            pl.BlockSpec((_F, _D), lambda t, f: (0, 0)),
            pl.BlockSpec((1, _D), lambda t, f: (0, 0)),
        ],
        out_specs=pl.BlockSpec((_TB, _D),
                               lambda t, f: (jnp.maximum(t - 1, 0), 0)),
        out_shape=jax.ShapeDtypeStruct((_T, _D), jnp.float32),
        scratch_shapes=[pltpu.VMEM((_TB, _F), jnp.float32),
                        pltpu.VMEM((_TB, _F), jnp.bfloat16)],
        compiler_params=pltpu.CompilerParams(
            dimension_semantics=("arbitrary", "arbitrary")),
    )(k_arr, hidden, W_enc, sqrtn, rsqrtn, wdec_bf16, b_dec.reshape(1, _D))

    return recon


# speedup vs baseline: 1.1232x; 1.1232x over previous
import functools

import jax
import jax.numpy as jnp
from jax.experimental import pallas as pl
from jax.experimental.pallas import tpu as pltpu

_D = 1024
_F = 8192
_T = 2048
_K = 50

_TB = 256
_FB = 2048
_NT = _T // _TB
_NF = _F // _FB


def _norms_body(wdec_ref, sn_ref, rn_ref, wb_ref):
    w = wdec_ref[...]
    sq = jnp.sum(w * w, axis=1)[None, :]
    sn_ref[...] = jnp.sqrt(sq)
    rn_ref[...] = jnp.where(sq > 0.0, jax.lax.rsqrt(sq), 0.0)
    wb_ref[...] = w.astype(jnp.bfloat16)


def _fused_body(k_ref, hid_ref, wenc_ref, sn_ref, rn_ref, wdec_ref, b_ref,
                out_ref, g_scr, sp_scr):
    t = pl.program_id(0)
    f = pl.program_id(1)

    @pl.when(t > 0)
    def _decode():
        acc = jnp.dot(sp_scr[:, pl.ds(f * _FB, _FB)],
                      wdec_ref[pl.ds(f * _FB, _FB), :],
                      preferred_element_type=jnp.float32)

        @pl.when(f == 0)
        def _init():
            out_ref[...] = acc

        @pl.when(f != 0)
        def _accum():
            out_ref[...] += acc

        @pl.when(f == _NF - 1)
        def _bias():
            out_ref[...] += b_ref[...]

    @pl.when(t < _NT)
    def _encode():
        a = jnp.maximum(jnp.dot(hid_ref[...], wenc_ref[...],
                                preferred_element_type=jnp.float32), 0.0)
        g_scr[:, pl.ds(f * _FB, _FB)] = a * sn_ref[...]

    @pl.when((t < _NT) & (f == _NF - 1))
    def _select():
        kk = jnp.minimum(k_ref[0], _K)
        g = g_scr[...]
        lo0 = jnp.zeros((_TB, 1), jnp.int32)
        hi0 = jax.lax.bitcast_convert_type(
            jnp.max(g, axis=1, keepdims=True), jnp.int32)
        clo0 = jnp.full((_TB, 1), _F, jnp.int32)
        chi0 = jnp.ones((_TB, 1), jnp.int32)
        lk = jnp.log(jnp.maximum(kk, 1).astype(jnp.float32))

        def done(lo, hi, clo):
            return (clo == kk) | (hi - lo <= 1)

        def cond(carry):
            lo, hi, clo, chi = carry
            return jnp.logical_not(jnp.all(done(lo, hi, clo)))

        def body(carry):
            lo, hi, clo, chi = carry
            frozen = done(lo, hi, clo)
            bise = lo + jax.lax.div(hi - lo, 2)
            vlo = jax.lax.bitcast_convert_type(lo, jnp.float32)
            vhi = jax.lax.bitcast_convert_type(hi, jnp.float32)
            llo = jnp.log(clo.astype(jnp.float32))
            lhi = jnp.log(jnp.maximum(chi, 1).astype(jnp.float32))
            frac = (llo - lk) / jnp.maximum(llo - lhi, 1e-6)
            vmid = vlo + (vhi - vlo) * frac
            imid = jnp.clip(jax.lax.bitcast_convert_type(vmid, jnp.int32),
                            lo + 1, hi - 1)
            m1 = jnp.minimum(imid, bise)
            m2 = jnp.maximum(imid, bise)
            v1 = jax.lax.bitcast_convert_type(m1, jnp.float32)
            v2 = jax.lax.bitcast_convert_type(m2, jnp.float32)
            c1 = jnp.sum((g >= v1).astype(jnp.int32), axis=1, keepdims=True)
            c2 = jnp.sum((g >= v2).astype(jnp.int32), axis=1, keepdims=True)
            ok2 = c2 >= kk
            ok1 = c1 >= kk
            nlo = jnp.where(ok2, m2, jnp.where(ok1, m1, lo))
            nclo = jnp.where(ok2, c2, jnp.where(ok1, c1, clo))
            nhi = jnp.where(ok2, hi, jnp.where(ok1, m2, m1))
            nchi = jnp.where(ok2, chi, jnp.where(ok1, c2, c1))
            return (jnp.where(frozen, lo, nlo), jnp.where(frozen, hi, nhi),
                    jnp.where(frozen, clo, nclo), jnp.where(frozen, chi, nchi))

        lo, _, _, _ = jax.lax.while_loop(cond, body, (lo0, hi0, clo0, chi0))
        vthr = jax.lax.bitcast_convert_type(lo, jnp.float32)
        sp_scr[...] = jnp.where(g >= vthr, g * rn_ref[...],
                                0.0).astype(jnp.bfloat16)


@functools.partial(jax.jit, static_argnames=())
def kernel(hidden, W_enc, W_dec, b_dec, k):
    k_arr = jnp.asarray(k, jnp.int32).reshape((1,))

    sqrtn, rsqrtn, wdec_bf16 = pl.pallas_call(
        _norms_body,
        grid=(_NF,),
        in_specs=[pl.BlockSpec((_FB, _D), lambda f: (f, 0))],
        out_specs=[pl.BlockSpec((1, _FB), lambda f: (0, f)),
                   pl.BlockSpec((1, _FB), lambda f: (0, f)),
                   pl.BlockSpec((_FB, _D), lambda f: (f, 0))],
        out_shape=[jax.ShapeDtypeStruct((1, _F), jnp.float32),
                   jax.ShapeDtypeStruct((1, _F), jnp.float32),
                   jax.ShapeDtypeStruct((_F, _D), jnp.bfloat16)],
    )(W_dec)

    recon = pl.pallas_call(
        _fused_body,
        grid=(_NT + 1, _NF),
        in_specs=[
            pl.BlockSpec(memory_space=pltpu.SMEM),
            pl.BlockSpec((_TB, _D), lambda t, f: (jnp.minimum(t, _NT - 1), 0)),
            pl.BlockSpec((_D, _FB), lambda t, f: (0, f)),
            pl.BlockSpec((1, _FB), lambda t, f: (0, f)),
            pl.BlockSpec((1, _F), lambda t, f: (0, 0)),
            pl.BlockSpec((_F, _D), lambda t, f: (0, 0)),
            pl.BlockSpec((1, _D), lambda t, f: (0, 0)),
        ],
        out_specs=pl.BlockSpec((_TB, _D),
                               lambda t, f: (jnp.maximum(t - 1, 0), 0)),
        out_shape=jax.ShapeDtypeStruct((_T, _D), jnp.float32),
        scratch_shapes=[pltpu.VMEM((_TB, _F), jnp.float32),
                        pltpu.VMEM((_TB, _F), jnp.bfloat16)],
        compiler_params=pltpu.CompilerParams(
            dimension_semantics=("arbitrary", "arbitrary")),
    )(k_arr, hidden, W_enc, sqrtn, rsqrtn, wdec_bf16, b_dec.reshape(1, _D))

    return recon
